# baseline (speedup 1.0000x reference)
import functools

import jax
import jax.numpy as jnp
from jax import lax
from jax.experimental import pallas as pl
from jax.experimental.pallas import tpu as pltpu
from jax.experimental.pallas import tpu_sc as plsc

CH = 128


def _dot_chunk(zb, wb, bb, ob):
    lane = lax.iota(jnp.int32, 16)
    last = lane == 15

    @plsc.parallel_loop(0, CH, 1, unroll=1)
    def body(i):
        acc = zb[i, pl.ds(0, 16)] * wb[i, pl.ds(0, 16)]
        for k in range(1, 8):
            acc = acc + zb[i, pl.ds(16 * k, 16)] * wb[i, pl.ds(16 * k, 16)]
        tot = plsc.cumsum(acc)
        plsc.store_scatter(ob, [jnp.full((16,), i, jnp.int32)], tot,
                           mask=last)
    for g in range(CH // 16):
        sl = pl.ds(g * 16, 16)
        ob[sl] = ob[sl] + bb[sl]


def kernel(Z, y, W1, b1):
    B, D = Z.shape
    info = plsc.get_sparse_core_info()
    nsub = info.num_subcores
    nw = info.num_cores * nsub
    bpw = B // nw
    nch = bpw // CH
    y32 = y.astype(jnp.int32)

    mesh = plsc.VectorSubcoreMesh(core_axis_name="c", subcore_axis_name="s")

    @functools.partial(
        pl.kernel,
        out_type=jax.ShapeDtypeStruct((B,), jnp.float32),
        mesh=mesh,
        compiler_params=pltpu.CompilerParams(needs_layout_passes=False),
        scratch_types=[
            pltpu.VMEM((bpw,), jnp.int32),
            pltpu.VMEM((2, CH, D), jnp.float32),
            pltpu.VMEM((nch, CH, D), jnp.float32),
            pltpu.VMEM((nch, CH), jnp.float32),
            pltpu.VMEM((nch, CH), jnp.float32),
            pltpu.SemaphoreType.DMA,
            pltpu.SemaphoreType.DMA,
            pltpu.SemaphoreType.DMA,
            pltpu.SemaphoreType.DMA,
            pltpu.SemaphoreType.DMA,
            pltpu.SemaphoreType.DMA,
            pltpu.SemaphoreType.DMA,
            pltpu.SemaphoreType.DMA,
            pltpu.SemaphoreType.DMA,
            pltpu.SemaphoreType.DMA,
            pltpu.SemaphoreType.DMA,
        ],
    )
    def k(z_hbm, y_hbm, w_hbm, b_hbm, out_hbm,
          idx_v, zbuf, wbuf, bbuf, obuf, *sems):
        wid = lax.axis_index("c") * nsub + lax.axis_index("s")
        base0 = wid * bpw
        pltpu.sync_copy(y_hbm.at[pl.ds(base0, bpw)], idx_v)

        ws, bs = [], []
        for c in range(nch):
            isl = idx_v.at[pl.ds(c * CH, CH)]
            ws.append(pltpu.async_copy(w_hbm.at[isl], wbuf.at[c],
                                       sems[2 + c]))
            bs.append(pltpu.async_copy(b_hbm.at[isl], bbuf.at[c],
                                       sems[6 + c]))

        def start_z(c, buf):
            return pltpu.async_copy(z_hbm.at[pl.ds(base0 + c * CH, CH)],
                                    zbuf.at[buf], sems[buf])

        hz = start_z(0, 0)
        outs = []
        for c in range(nch):
            buf = c & 1
            nz = start_z(c + 1, 1 - buf) if c + 1 < nch else None
            hz.wait()
            ws[c].wait()
            bs[c].wait()
            _dot_chunk(zbuf.at[buf], wbuf.at[c], bbuf.at[c], obuf.at[c])
            outs.append(pltpu.async_copy(
                obuf.at[c], out_hbm.at[pl.ds(base0 + c * CH, CH)], sems[10]))
            hz = nz
        for h in outs:
            h.wait()

    return k(Z, y32, W1, b1)


# speedup vs baseline: 1.0409x; 1.0409x over previous
import functools

import jax
import jax.numpy as jnp
from jax import lax
from jax.experimental import pallas as pl
from jax.experimental.pallas import tpu as pltpu
from jax.experimental.pallas import tpu_sc as plsc

CH = 128


def _dot_chunk(zb, wb, bb, ob):
    lane = lax.iota(jnp.int32, 16)
    last = lane == 15

    @plsc.parallel_loop(0, CH, 1, unroll=1)
    def body(i):
        acc = zb[i, pl.ds(0, 16)] * wb[i, pl.ds(0, 16)]
        for k in range(1, 8):
            acc = acc + zb[i, pl.ds(16 * k, 16)] * wb[i, pl.ds(16 * k, 16)]
        tot = plsc.cumsum(acc)
        plsc.store_scatter(ob, [jnp.full((16,), i, jnp.int32)], tot,
                           mask=last)
    for g in range(CH // 16):
        sl = pl.ds(g * 16, 16)
        ob[sl] = ob[sl] + bb[sl]


def kernel(Z, y, W1, b1):
    B, D = Z.shape
    info = plsc.get_sparse_core_info()
    nsub = info.num_subcores
    nw = info.num_cores * nsub
    bpw = B // nw
    nch = bpw // CH
    y32 = y.astype(jnp.int32)

    mesh = plsc.VectorSubcoreMesh(core_axis_name="c", subcore_axis_name="s")

    @functools.partial(
        pl.kernel,
        out_type=jax.ShapeDtypeStruct((B,), jnp.float32),
        mesh=mesh,
        compiler_params=pltpu.CompilerParams(needs_layout_passes=False),
        scratch_types=[
            pltpu.VMEM((bpw,), jnp.int32),
            pltpu.VMEM((2, CH, D), jnp.float32),
            pltpu.VMEM((2, CH, D), jnp.float32),
            pltpu.VMEM((2, CH), jnp.float32),
            pltpu.VMEM((nch, CH), jnp.float32),
            pltpu.SemaphoreType.DMA,
            pltpu.SemaphoreType.DMA,
            pltpu.SemaphoreType.DMA,
            pltpu.SemaphoreType.DMA,
            pltpu.SemaphoreType.DMA,
            pltpu.SemaphoreType.DMA,
            pltpu.SemaphoreType.DMA,
            pltpu.SemaphoreType.DMA,
            pltpu.SemaphoreType.DMA,
            pltpu.SemaphoreType.DMA,
            pltpu.SemaphoreType.DMA,
        ],
    )
    def k(z_hbm, y_hbm, w_hbm, b_hbm, out_hbm,
          idx_v, zbuf, wbuf, bbuf, obuf, *sems):
        wid = lax.axis_index("c") * nsub + lax.axis_index("s")
        base0 = wid * bpw
        pltpu.sync_copy(y_hbm.at[pl.ds(base0, bpw)], idx_v)

        def start(c, buf):
            isl = idx_v.at[pl.ds(c * CH, CH)]
            hz = pltpu.async_copy(z_hbm.at[pl.ds(base0 + c * CH, CH)],
                                  zbuf.at[buf], sems[buf])
            hw = pltpu.async_copy(w_hbm.at[isl], wbuf.at[buf], sems[2 + buf])
            hb = pltpu.async_copy(b_hbm.at[isl], bbuf.at[buf], sems[4 + buf])
            return (hz, hw, hb)

        hs = start(0, 0)
        outs = []
        for c in range(nch):
            buf = c & 1
            nxt = start(c + 1, 1 - buf) if c + 1 < nch else None
            for h in hs:
                h.wait()
            _dot_chunk(zbuf.at[buf], wbuf.at[buf], bbuf.at[buf], obuf.at[c])
            outs.append(pltpu.async_copy(
                obuf.at[c], out_hbm.at[pl.ds(base0 + c * CH, CH)], sems[10]))
            hs = nxt
        for h in outs:
            h.wait()

    return k(Z, y32, W1, b1)


# speedup vs baseline: 1.0457x; 1.0046x over previous
import functools

import jax
import jax.numpy as jnp
from jax import lax
from jax.experimental import pallas as pl
from jax.experimental.pallas import tpu as pltpu
from jax.experimental.pallas import tpu_sc as plsc

CH = 128


def _dot_chunk(zb, wb, bb, ob):
    lane = lax.iota(jnp.int32, 16)
    last = lane == 15

    @plsc.parallel_loop(0, CH, 1, unroll=1)
    def body(i):
        acc = zb[i, pl.ds(0, 16)] * wb[i, pl.ds(0, 16)]
        for k in range(1, 8):
            acc = acc + zb[i, pl.ds(16 * k, 16)] * wb[i, pl.ds(16 * k, 16)]
        tot = plsc.cumsum(acc)
        plsc.store_scatter(ob, [jnp.full((16,), i, jnp.int32)], tot,
                           mask=last)
    for g in range(CH // 16):
        sl = pl.ds(g * 16, 16)
        ob[sl] = ob[sl] + bb[sl]


def kernel(Z, y, W1, b1):
    B, D = Z.shape
    info = plsc.get_sparse_core_info()
    nsub = info.num_subcores
    nw = info.num_cores * nsub
    bpw = B // nw
    nch = bpw // CH
    y32 = y.astype(jnp.int32)

    mesh = plsc.VectorSubcoreMesh(core_axis_name="c", subcore_axis_name="s")

    @functools.partial(
        pl.kernel,
        out_type=jax.ShapeDtypeStruct((B,), jnp.float32),
        mesh=mesh,
        compiler_params=pltpu.CompilerParams(needs_layout_passes=False),
        scratch_types=[
            pltpu.VMEM((bpw,), jnp.int32),
            pltpu.VMEM((2, CH, D), jnp.float32),
            pltpu.VMEM((2, CH, D), jnp.float32),
            pltpu.VMEM((2, CH), jnp.float32),
            pltpu.VMEM((nch, CH), jnp.float32),
            pltpu.SemaphoreType.DMA,
            pltpu.SemaphoreType.DMA,
            pltpu.SemaphoreType.DMA,
            pltpu.SemaphoreType.DMA,
            pltpu.SemaphoreType.DMA,
            pltpu.SemaphoreType.DMA,
            pltpu.SemaphoreType.DMA,
            pltpu.SemaphoreType.DMA,
            pltpu.SemaphoreType.DMA,
            pltpu.SemaphoreType.DMA,
            pltpu.SemaphoreType.DMA,
        ],
    )
    def k(z_hbm, y_hbm, w_hbm, b_hbm, out_hbm,
          idx_v, zbuf, wbuf, bbuf, obuf, *sems):
        wid = lax.axis_index("c") * nsub + lax.axis_index("s")
        base0 = wid * bpw
        hy = pltpu.async_copy(y_hbm.at[pl.ds(base0, bpw)], idx_v, sems[9])

        def start_z(c, buf):
            return pltpu.async_copy(z_hbm.at[pl.ds(base0 + c * CH, CH)],
                                    zbuf.at[buf], sems[buf])

        def start_wb(c, buf):
            isl = idx_v.at[pl.ds(c * CH, CH)]
            hw = pltpu.async_copy(w_hbm.at[isl], wbuf.at[buf], sems[2 + buf])
            hb = pltpu.async_copy(b_hbm.at[isl], bbuf.at[buf], sems[4 + buf])
            return (hw, hb)

        hz = start_z(0, 0)
        hz2 = start_z(1, 1)
        hy.wait()
        hs = (hz,) + start_wb(0, 0)
        outs = []
        for c in range(nch):
            buf = c & 1
            if c + 1 < nch:
                nz = hz2 if c == 0 else start_z(c + 1, 1 - buf)
                nxt = (nz,) + start_wb(c + 1, 1 - buf)
            else:
                nxt = None
            for h in hs:
                h.wait()
            _dot_chunk(zbuf.at[buf], wbuf.at[buf], bbuf.at[buf], obuf.at[c])
            outs.append(pltpu.async_copy(
                obuf.at[c], out_hbm.at[pl.ds(base0 + c * CH, CH)], sems[10]))
            hs = nxt
        for h in outs:
            h.wait()

    return k(Z, y32, W1, b1)
